# Initial kernel scaffold; baseline (speedup 1.0000x reference)
import jax
import jax.numpy as jnp
from jax.experimental import pallas as pl


def kernel(x, W, b, gamma, beta):
    raise NotImplementedError("write your pallas kernel here")


# speedup vs baseline: 8.1446x; 8.1446x over previous
import functools

import jax
import jax.numpy as jnp
from jax import lax
from jax.experimental import pallas as pl
from jax.experimental.pallas import tpu as pltpu
from jax.experimental.pallas import tpu_sc as plsc

K_NN = 20
RB = 256


def _phase_a_body(x_ref, wd_ref, w2_ref, bias_ref, idx_ref, u_ref, v_ref):
    b = pl.program_id(0)
    i = pl.program_id(1)
    xb = x_ref[0]
    F, N = xb.shape
    xr = x_ref[0, :, pl.ds(i * RB, RB)]
    xs = jnp.sum(xb * xb, axis=0)
    xsr = jnp.sum(xr * xr, axis=0)
    g = lax.dot_general(xr, xb, (((0,), (0,)), ((), ())),
                        preferred_element_type=jnp.float32)
    d = xsr[:, None] + xs[None, :] - 2.0 * g
    iota = lax.broadcasted_iota(jnp.int32, (RB, N), 1)
    off = b * N

    def body(t, d):
        mn = jnp.min(d, axis=1, keepdims=True)
        am = jnp.min(jnp.where(d == mn, iota, N), axis=1)
        idx_ref[0, pl.ds(t, 1), :] = (am + off)[None, :]
        return jnp.where(iota == am[:, None], jnp.inf, d)

    lax.fori_loop(0, K_NN + 1, body, d)

    u_ref[0] = lax.dot_general(xr, wd_ref[...], (((0,), (1,)), ((), ())),
                               preferred_element_type=jnp.float32) \
        + bias_ref[0][None, :]
    v_ref[0] = lax.dot_general(xr, w2_ref[...], (((0,), (1,)), ((), ())),
                               preferred_element_type=jnp.float32)


def _phase_a(x, wd, w2, bias, interpret=False):
    B, F, N = x.shape
    Fout = wd.shape[0]
    return pl.pallas_call(
        _phase_a_body,
        grid=(B, N // RB),
        in_specs=[
            pl.BlockSpec((1, F, N), lambda b, i: (b, 0, 0)),
            pl.BlockSpec((Fout, F), lambda b, i: (0, 0)),
            pl.BlockSpec((Fout, F), lambda b, i: (0, 0)),
            pl.BlockSpec((1, Fout), lambda b, i: (0, 0)),
        ],
        out_specs=[
            pl.BlockSpec((1, K_NN + 1, RB), lambda b, i: (b, 0, i)),
            pl.BlockSpec((1, RB, Fout), lambda b, i: (b, i, 0)),
            pl.BlockSpec((1, RB, Fout), lambda b, i: (b, i, 0)),
        ],
        out_shape=[
            jax.ShapeDtypeStruct((B, K_NN + 1, N), jnp.int32),
            jax.ShapeDtypeStruct((B, N, Fout), jnp.float32),
            jax.ShapeDtypeStruct((B, N, Fout), jnp.float32),
        ],
        interpret=interpret,
    )(x, wd, w2, bias)


_SC_CORES = 2
_SC_SUBCORES = 16
_NW = _SC_CORES * _SC_SUBCORES
_CH = 64


def _phase_b(vg, idxg):
    BN, F = vg.shape
    B, K1, N = idxg.shape
    npw = BN // _NW
    nchunk = npw // _CH
    mesh = plsc.VectorSubcoreMesh(core_axis_name="c", subcore_axis_name="s")

    @functools.partial(
        pl.kernel,
        out_type=[jax.ShapeDtypeStruct((BN, F), jnp.float32)] * 3,
        mesh=mesh,
        scratch_types=[
            pltpu.VMEM((K1, npw), jnp.int32),
            pltpu.VMEM((K_NN, _CH, F), jnp.float32),
            pltpu.VMEM((_CH, F), jnp.float32),
            pltpu.VMEM((_CH, F), jnp.float32),
            pltpu.VMEM((_CH, F), jnp.float32),
            pltpu.SemaphoreType.DMA,
        ],
        compiler_params=pltpu.CompilerParams(use_tc_tiling_on_sc=False),
    )
    def kern(vg_hbm, idx_hbm, mx_hbm, sm_hbm, sq_hbm,
             idxb, rows, om, osum, osq, sem):
        wid = lax.axis_index("s") * _SC_CORES + lax.axis_index("c")
        g0 = wid * npw
        bb = g0 // N
        n0 = g0 % N
        pltpu.sync_copy(idx_hbm.at[bb, :, pl.ds(n0, npw)], idxb)
        for chunk in range(nchunk):
            base = chunk * _CH
            descs = [
                pltpu.async_copy(
                    vg_hbm.at[idxb.at[t + 1, pl.ds(base, _CH)]],
                    rows.at[t], sem)
                for t in range(K_NN)
            ]
            for dsc in descs:
                dsc.wait()

            def nnbody(nn, carry):
                for c in range(F // 16):
                    sl = pl.ds(c * 16, 16)
                    r = rows[0, nn, sl]
                    m = r
                    s = r
                    q = r * r
                    for t in range(1, K_NN):
                        r = rows[t, nn, sl]
                        m = jnp.maximum(m, r)
                        s = s + r
                        q = q + r * r
                    om[nn, sl] = m
                    osum[nn, sl] = s
                    osq[nn, sl] = q
                return carry

            lax.fori_loop(0, _CH, nnbody, 0)
            pltpu.sync_copy(om, mx_hbm.at[pl.ds(g0 + base, _CH)])
            pltpu.sync_copy(osum, sm_hbm.at[pl.ds(g0 + base, _CH)])
            pltpu.sync_copy(osq, sq_hbm.at[pl.ds(g0 + base, _CH)])

    return kern(vg, idxg)


def _phase_c1_body(u_ref, s_ref, q_ref, s1_ref, s2_ref):
    i = pl.program_id(0)

    @pl.when(i == 0)
    def _():
        s1_ref[...] = jnp.zeros_like(s1_ref)
        s2_ref[...] = jnp.zeros_like(s2_ref)

    u = u_ref[...]
    s = s_ref[...]
    q = q_ref[...]
    k = jnp.float32(K_NN)
    y1 = jnp.sum(k * u + s, axis=0)
    y2 = jnp.sum(k * (u * u) + 2.0 * (u * s) + q, axis=0)
    s1_ref[...] = s1_ref[...] + y1[None, :]
    s2_ref[...] = s2_ref[...] + y2[None, :]


def _phase_c1(u2, sm, sq, interpret=False):
    BN, F = u2.shape
    return pl.pallas_call(
        _phase_c1_body,
        grid=(BN // RB,),
        in_specs=[pl.BlockSpec((RB, F), lambda i: (i, 0))] * 3,
        out_specs=[pl.BlockSpec((1, F), lambda i: (0, 0))] * 2,
        out_shape=[jax.ShapeDtypeStruct((1, F), jnp.float32)] * 2,
        interpret=interpret,
    )(u2, sm, sq)


def _phase_c2_body(n_total, u_ref, mx_ref, s1_ref, s2_ref, g_ref, be_ref,
                   out_ref):
    mean = s1_ref[0] / n_total
    var = s2_ref[0] / n_total - mean * mean
    inv = lax.rsqrt(var + 1e-5)
    scale = g_ref[0] * inv
    z = (u_ref[0] + mx_ref[...] - mean[None, :]) * scale[None, :] \
        + be_ref[0][None, :]
    z = jnp.maximum(z, 0.0)
    out_ref[0] = z.T


def _phase_c2(u, mx, s1, s2, gamma, beta, interpret=False):
    B, N, F = u.shape
    n_total = float(B * N * K_NN)
    return pl.pallas_call(
        functools.partial(_phase_c2_body, n_total),
        grid=(B, N // RB),
        in_specs=[
            pl.BlockSpec((1, RB, F), lambda b, i: (b, i, 0)),
            pl.BlockSpec((RB, F), lambda b, i: (b * (N // RB) + i, 0)),
            pl.BlockSpec((1, F), lambda b, i: (0, 0)),
            pl.BlockSpec((1, F), lambda b, i: (0, 0)),
            pl.BlockSpec((1, F), lambda b, i: (0, 0)),
            pl.BlockSpec((1, F), lambda b, i: (0, 0)),
        ],
        out_specs=pl.BlockSpec((1, F, RB), lambda b, i: (b, 0, i)),
        out_shape=jax.ShapeDtypeStruct((B, F, N), jnp.float32),
        interpret=interpret,
    )(u, mx, s1, s2, gamma, beta)


@jax.jit
def kernel(x, W, b, gamma, beta):
    B, F, N = x.shape
    Fout = W.shape[0]
    w1 = W[:, :F]
    w2 = W[:, F:]
    wd = w1 - w2
    idxg, u, v = _phase_a(x, wd, w2, b.reshape(1, Fout))
    vg = v.reshape(B * N, Fout)
    mx, sm, sq = _phase_b(vg, idxg)
    s1, s2 = _phase_c1(u.reshape(B * N, Fout), sm, sq)
    return _phase_c2(u, mx, s1, s2,
                     gamma.reshape(1, Fout), beta.reshape(1, Fout))


# speedup vs baseline: 13.4624x; 1.6529x over previous
import functools

import jax
import jax.numpy as jnp
from jax import lax
from jax.experimental import pallas as pl
from jax.experimental.pallas import tpu as pltpu
from jax.experimental.pallas import tpu_sc as plsc

K_NN = 20
RB = 512


def _phase_a_body(x_ref, wd_ref, w2_ref, bias_ref, idx_ref, u_ref, v_ref):
    b = pl.program_id(0)
    i = pl.program_id(1)
    xb = x_ref[0]
    F, N = xb.shape
    xr = x_ref[0, :, pl.ds(i * RB, RB)]
    xs = jnp.sum(xb * xb, axis=0)
    xsr = jnp.sum(xr * xr, axis=0)
    g = lax.dot_general(xr, xb, (((0,), (0,)), ((), ())),
                        preferred_element_type=jnp.float32)
    d = xsr[:, None] + xs[None, :] - 2.0 * g
    iota = lax.broadcasted_iota(jnp.int32, (RB, N), 1)
    off = b * N

    def body(t, carry):
        mn, acc = carry
        am = jnp.min(jnp.where(d == mn, iota, N), axis=1)
        mn_next = jnp.min(jnp.where(d > mn, d, jnp.inf), axis=1,
                          keepdims=True)
        acc = jnp.where(lane32 == t, (am + off)[:, None], acc)
        return mn_next, acc

    lane32 = lax.broadcasted_iota(jnp.int32, (RB, 32), 1)
    acc0 = jnp.zeros((RB, 32), jnp.int32)
    mn0 = jnp.min(d, axis=1, keepdims=True)
    _, acc = lax.fori_loop(0, K_NN + 1, body, (mn0, acc0))
    idx_ref[0] = acc.T[:K_NN + 1, :]

    u_ref[0] = lax.dot_general(xr, wd_ref[...], (((0,), (1,)), ((), ())),
                               preferred_element_type=jnp.float32) \
        + bias_ref[0][None, :]
    v_ref[0] = lax.dot_general(xr, w2_ref[...], (((0,), (1,)), ((), ())),
                               preferred_element_type=jnp.float32)


def _phase_a(x, wd, w2, bias, interpret=False):
    B, F, N = x.shape
    Fout = wd.shape[0]
    return pl.pallas_call(
        _phase_a_body,
        grid=(B, N // RB),
        in_specs=[
            pl.BlockSpec((1, F, N), lambda b, i: (b, 0, 0)),
            pl.BlockSpec((Fout, F), lambda b, i: (0, 0)),
            pl.BlockSpec((Fout, F), lambda b, i: (0, 0)),
            pl.BlockSpec((1, Fout), lambda b, i: (0, 0)),
        ],
        out_specs=[
            pl.BlockSpec((1, K_NN + 1, RB), lambda b, i: (b, 0, i)),
            pl.BlockSpec((1, RB, Fout), lambda b, i: (b, i, 0)),
            pl.BlockSpec((1, RB, Fout), lambda b, i: (b, i, 0)),
        ],
        out_shape=[
            jax.ShapeDtypeStruct((B, K_NN + 1, N), jnp.int32),
            jax.ShapeDtypeStruct((B, N, Fout), jnp.float32),
            jax.ShapeDtypeStruct((B, N, Fout), jnp.float32),
        ],
        interpret=interpret,
    )(x, wd, w2, bias)


_SC_CORES = 2
_SC_SUBCORES = 16
_NW = _SC_CORES * _SC_SUBCORES
_CH = 64


def _phase_b(vg, idxg):
    BN, F = vg.shape
    B, K1, N = idxg.shape
    npw = BN // _NW
    nchunk = npw // _CH
    mesh = plsc.VectorSubcoreMesh(core_axis_name="c", subcore_axis_name="s")

    @functools.partial(
        pl.kernel,
        out_type=[jax.ShapeDtypeStruct((BN, F), jnp.float32)] * 3,
        mesh=mesh,
        scratch_types=[
            pltpu.VMEM((K1, npw), jnp.int32),
            pltpu.VMEM((K_NN, _CH, F), jnp.float32),
            pltpu.VMEM((_CH, F), jnp.float32),
            pltpu.VMEM((_CH, F), jnp.float32),
            pltpu.VMEM((_CH, F), jnp.float32),
            pltpu.SemaphoreType.DMA,
        ],
        compiler_params=pltpu.CompilerParams(use_tc_tiling_on_sc=False),
    )
    def kern(vg_hbm, idx_hbm, mx_hbm, sm_hbm, sq_hbm,
             idxb, rows, om, osum, osq, sem):
        wid = lax.axis_index("s") * _SC_CORES + lax.axis_index("c")
        g0 = wid * npw
        bb = g0 // N
        n0 = g0 % N
        pltpu.sync_copy(idx_hbm.at[bb, :, pl.ds(n0, npw)], idxb)
        for chunk in range(nchunk):
            base = chunk * _CH
            descs = [
                pltpu.async_copy(
                    vg_hbm.at[idxb.at[t + 1, pl.ds(base, _CH)]],
                    rows.at[t], sem)
                for t in range(K_NN)
            ]
            for dsc in descs:
                dsc.wait()

            def nnbody(nn, carry):
                for c in range(F // 16):
                    sl = pl.ds(c * 16, 16)
                    r = rows[0, nn, sl]
                    m = r
                    s = r
                    q = r * r
                    for t in range(1, K_NN):
                        r = rows[t, nn, sl]
                        m = jnp.maximum(m, r)
                        s = s + r
                        q = q + r * r
                    om[nn, sl] = m
                    osum[nn, sl] = s
                    osq[nn, sl] = q
                return carry

            lax.fori_loop(0, _CH, nnbody, 0)
            pltpu.sync_copy(om, mx_hbm.at[pl.ds(g0 + base, _CH)])
            pltpu.sync_copy(osum, sm_hbm.at[pl.ds(g0 + base, _CH)])
            pltpu.sync_copy(osq, sq_hbm.at[pl.ds(g0 + base, _CH)])

    return kern(vg, idxg)


def _phase_c1_body(u_ref, s_ref, q_ref, s1_ref, s2_ref):
    i = pl.program_id(0)

    @pl.when(i == 0)
    def _():
        s1_ref[...] = jnp.zeros_like(s1_ref)
        s2_ref[...] = jnp.zeros_like(s2_ref)

    u = u_ref[...]
    s = s_ref[...]
    q = q_ref[...]
    k = jnp.float32(K_NN)
    y1 = jnp.sum(k * u + s, axis=0)
    y2 = jnp.sum(k * (u * u) + 2.0 * (u * s) + q, axis=0)
    s1_ref[...] = s1_ref[...] + y1[None, :]
    s2_ref[...] = s2_ref[...] + y2[None, :]


def _phase_c1(u2, sm, sq, interpret=False):
    BN, F = u2.shape
    return pl.pallas_call(
        _phase_c1_body,
        grid=(BN // RB,),
        in_specs=[pl.BlockSpec((RB, F), lambda i: (i, 0))] * 3,
        out_specs=[pl.BlockSpec((1, F), lambda i: (0, 0))] * 2,
        out_shape=[jax.ShapeDtypeStruct((1, F), jnp.float32)] * 2,
        interpret=interpret,
    )(u2, sm, sq)


def _phase_c2_body(n_total, u_ref, mx_ref, s1_ref, s2_ref, g_ref, be_ref,
                   out_ref):
    mean = s1_ref[0] / n_total
    var = s2_ref[0] / n_total - mean * mean
    inv = lax.rsqrt(var + 1e-5)
    scale = g_ref[0] * inv
    z = (u_ref[0] + mx_ref[...] - mean[None, :]) * scale[None, :] \
        + be_ref[0][None, :]
    z = jnp.maximum(z, 0.0)
    out_ref[0] = z.T


def _phase_c2(u, mx, s1, s2, gamma, beta, interpret=False):
    B, N, F = u.shape
    n_total = float(B * N * K_NN)
    return pl.pallas_call(
        functools.partial(_phase_c2_body, n_total),
        grid=(B, N // RB),
        in_specs=[
            pl.BlockSpec((1, RB, F), lambda b, i: (b, i, 0)),
            pl.BlockSpec((RB, F), lambda b, i: (b * (N // RB) + i, 0)),
            pl.BlockSpec((1, F), lambda b, i: (0, 0)),
            pl.BlockSpec((1, F), lambda b, i: (0, 0)),
            pl.BlockSpec((1, F), lambda b, i: (0, 0)),
            pl.BlockSpec((1, F), lambda b, i: (0, 0)),
        ],
        out_specs=pl.BlockSpec((1, F, RB), lambda b, i: (b, 0, i)),
        out_shape=jax.ShapeDtypeStruct((B, F, N), jnp.float32),
        interpret=interpret,
    )(u, mx, s1, s2, gamma, beta)


@jax.jit
def kernel(x, W, b, gamma, beta):
    B, F, N = x.shape
    Fout = W.shape[0]
    w1 = W[:, :F]
    w2 = W[:, F:]
    wd = w1 - w2
    idxg, u, v = _phase_a(x, wd, w2, b.reshape(1, Fout))
    vg = v.reshape(B * N, Fout)
    mx, sm, sq = _phase_b(vg, idxg)
    s1, s2 = _phase_c1(u.reshape(B * N, Fout), sm, sq)
    return _phase_c2(u, mx, s1, s2,
                     gamma.reshape(1, Fout), beta.reshape(1, Fout))


# speedup vs baseline: 14.4564x; 1.0738x over previous
import functools

import jax
import jax.numpy as jnp
from jax import lax
from jax.experimental import pallas as pl
from jax.experimental.pallas import tpu as pltpu
from jax.experimental.pallas import tpu_sc as plsc

K_NN = 20
RB = 1024


def _phase_a_body(x_ref, wd_ref, w2_ref, bias_ref, idx_ref, u_ref, v_ref):
    b = pl.program_id(0)
    i = pl.program_id(1)
    xb = x_ref[0]
    F, N = xb.shape
    xr = x_ref[0, :, pl.ds(i * RB, RB)]
    xs = jnp.sum(xb * xb, axis=0)
    xsr = jnp.sum(xr * xr, axis=0)
    g = lax.dot_general(xr, xb, (((0,), (0,)), ((), ())),
                        preferred_element_type=jnp.float32)
    d = xsr[:, None] + xs[None, :] - 2.0 * g
    iota = lax.broadcasted_iota(jnp.int32, (RB, N), 1)
    off = b * N

    def body(t, carry):
        mn, acc = carry
        am = jnp.min(jnp.where(d == mn, iota, N), axis=1)
        mn_next = jnp.min(jnp.where(d > mn, d, jnp.inf), axis=1,
                          keepdims=True)
        acc = jnp.where(lane32 == t, (am + off)[:, None], acc)
        return mn_next, acc

    lane32 = lax.broadcasted_iota(jnp.int32, (RB, 32), 1)
    acc0 = jnp.zeros((RB, 32), jnp.int32)
    mn0 = jnp.min(d, axis=1, keepdims=True)
    _, acc = lax.fori_loop(0, K_NN + 1, body, (mn0, acc0))
    idx_ref[0] = acc.T[:K_NN + 1, :]

    u_ref[0] = lax.dot_general(xr, wd_ref[...], (((0,), (1,)), ((), ())),
                               preferred_element_type=jnp.float32) \
        + bias_ref[0][None, :]
    v_ref[0] = lax.dot_general(xr, w2_ref[...], (((0,), (1,)), ((), ())),
                               preferred_element_type=jnp.float32)


def _phase_a(x, wd, w2, bias, interpret=False):
    B, F, N = x.shape
    Fout = wd.shape[0]
    return pl.pallas_call(
        _phase_a_body,
        grid=(B, N // RB),
        in_specs=[
            pl.BlockSpec((1, F, N), lambda b, i: (b, 0, 0)),
            pl.BlockSpec((Fout, F), lambda b, i: (0, 0)),
            pl.BlockSpec((Fout, F), lambda b, i: (0, 0)),
            pl.BlockSpec((1, Fout), lambda b, i: (0, 0)),
        ],
        out_specs=[
            pl.BlockSpec((1, K_NN + 1, RB), lambda b, i: (b, 0, i)),
            pl.BlockSpec((1, RB, Fout), lambda b, i: (b, i, 0)),
            pl.BlockSpec((1, RB, Fout), lambda b, i: (b, i, 0)),
        ],
        out_shape=[
            jax.ShapeDtypeStruct((B, K_NN + 1, N), jnp.int32),
            jax.ShapeDtypeStruct((B, N, Fout), jnp.float32),
            jax.ShapeDtypeStruct((B, N, Fout), jnp.float32),
        ],
        interpret=interpret,
    )(x, wd, w2, bias)


_SC_CORES = 2
_SC_SUBCORES = 16
_NW = _SC_CORES * _SC_SUBCORES
_CH = 64


def _phase_b(vg, idxg):
    BN, F = vg.shape
    B, K1, N = idxg.shape
    npw = BN // _NW
    nchunk = npw // _CH
    mesh = plsc.VectorSubcoreMesh(core_axis_name="c", subcore_axis_name="s")

    @functools.partial(
        pl.kernel,
        out_type=[jax.ShapeDtypeStruct((BN, F), jnp.float32)] * 3,
        mesh=mesh,
        scratch_types=[
            pltpu.VMEM((K1, npw), jnp.int32),
            pltpu.VMEM((K_NN, _CH, F), jnp.float32),
            pltpu.VMEM((_CH, F), jnp.float32),
            pltpu.VMEM((_CH, F), jnp.float32),
            pltpu.VMEM((_CH, F), jnp.float32),
            pltpu.SemaphoreType.DMA,
        ],
        compiler_params=pltpu.CompilerParams(use_tc_tiling_on_sc=False),
    )
    def kern(vg_hbm, idx_hbm, mx_hbm, sm_hbm, sq_hbm,
             idxb, rows, om, osum, osq, sem):
        wid = lax.axis_index("s") * _SC_CORES + lax.axis_index("c")
        g0 = wid * npw
        bb = g0 // N
        n0 = g0 % N
        pltpu.sync_copy(idx_hbm.at[bb, :, pl.ds(n0, npw)], idxb)
        for chunk in range(nchunk):
            base = chunk * _CH
            descs = [
                pltpu.async_copy(
                    vg_hbm.at[idxb.at[t + 1, pl.ds(base, _CH)]],
                    rows.at[t], sem)
                for t in range(K_NN)
            ]
            for dsc in descs:
                dsc.wait()

            def nnbody(nn, carry):
                for c in range(F // 16):
                    sl = pl.ds(c * 16, 16)
                    r = rows[0, nn, sl]
                    m = r
                    s = r
                    q = r * r
                    for t in range(1, K_NN):
                        r = rows[t, nn, sl]
                        m = jnp.maximum(m, r)
                        s = s + r
                        q = q + r * r
                    om[nn, sl] = m
                    osum[nn, sl] = s
                    osq[nn, sl] = q
                return carry

            lax.fori_loop(0, _CH, nnbody, 0)
            pltpu.sync_copy(om, mx_hbm.at[pl.ds(g0 + base, _CH)])
            pltpu.sync_copy(osum, sm_hbm.at[pl.ds(g0 + base, _CH)])
            pltpu.sync_copy(osq, sq_hbm.at[pl.ds(g0 + base, _CH)])

    return kern(vg, idxg)


def _phase_c1_body(u_ref, s_ref, q_ref, s1_ref, s2_ref):
    i = pl.program_id(0)

    @pl.when(i == 0)
    def _():
        s1_ref[...] = jnp.zeros_like(s1_ref)
        s2_ref[...] = jnp.zeros_like(s2_ref)

    u = u_ref[...]
    s = s_ref[...]
    q = q_ref[...]
    k = jnp.float32(K_NN)
    y1 = jnp.sum(k * u + s, axis=0)
    y2 = jnp.sum(k * (u * u) + 2.0 * (u * s) + q, axis=0)
    s1_ref[...] = s1_ref[...] + y1[None, :]
    s2_ref[...] = s2_ref[...] + y2[None, :]


def _phase_c1(u2, sm, sq, interpret=False):
    BN, F = u2.shape
    return pl.pallas_call(
        _phase_c1_body,
        grid=(BN // RB,),
        in_specs=[pl.BlockSpec((RB, F), lambda i: (i, 0))] * 3,
        out_specs=[pl.BlockSpec((1, F), lambda i: (0, 0))] * 2,
        out_shape=[jax.ShapeDtypeStruct((1, F), jnp.float32)] * 2,
        interpret=interpret,
    )(u2, sm, sq)


def _phase_c2_body(n_total, u_ref, mx_ref, s1_ref, s2_ref, g_ref, be_ref,
                   out_ref):
    mean = s1_ref[0] / n_total
    var = s2_ref[0] / n_total - mean * mean
    inv = lax.rsqrt(var + 1e-5)
    scale = g_ref[0] * inv
    z = (u_ref[0] + mx_ref[...] - mean[None, :]) * scale[None, :] \
        + be_ref[0][None, :]
    z = jnp.maximum(z, 0.0)
    out_ref[0] = z.T


def _phase_c2(u, mx, s1, s2, gamma, beta, interpret=False):
    B, N, F = u.shape
    n_total = float(B * N * K_NN)
    return pl.pallas_call(
        functools.partial(_phase_c2_body, n_total),
        grid=(B, N // RB),
        in_specs=[
            pl.BlockSpec((1, RB, F), lambda b, i: (b, i, 0)),
            pl.BlockSpec((RB, F), lambda b, i: (b * (N // RB) + i, 0)),
            pl.BlockSpec((1, F), lambda b, i: (0, 0)),
            pl.BlockSpec((1, F), lambda b, i: (0, 0)),
            pl.BlockSpec((1, F), lambda b, i: (0, 0)),
            pl.BlockSpec((1, F), lambda b, i: (0, 0)),
        ],
        out_specs=pl.BlockSpec((1, F, RB), lambda b, i: (b, 0, i)),
        out_shape=jax.ShapeDtypeStruct((B, F, N), jnp.float32),
        interpret=interpret,
    )(u, mx, s1, s2, gamma, beta)


@jax.jit
def kernel(x, W, b, gamma, beta):
    B, F, N = x.shape
    Fout = W.shape[0]
    w1 = W[:, :F]
    w2 = W[:, F:]
    wd = w1 - w2
    idxg, u, v = _phase_a(x, wd, w2, b.reshape(1, Fout))
    vg = v.reshape(B * N, Fout)
    mx, sm, sq = _phase_b(vg, idxg)
    s1, s2 = _phase_c1(u.reshape(B * N, Fout), sm, sq)
    return _phase_c2(u, mx, s1, s2,
                     gamma.reshape(1, Fout), beta.reshape(1, Fout))


# speedup vs baseline: 14.8555x; 1.0276x over previous
import functools

import jax
import jax.numpy as jnp
from jax import lax
from jax.experimental import pallas as pl
from jax.experimental.pallas import tpu as pltpu
from jax.experimental.pallas import tpu_sc as plsc

K_NN = 20
RB = 2048


def _phase_a_body(x_ref, wd_ref, w2_ref, bias_ref, idx_ref, u_ref, v_ref):
    b = pl.program_id(0)
    i = pl.program_id(1)
    xb = x_ref[0]
    F, N = xb.shape
    xr = x_ref[0, :, pl.ds(i * RB, RB)]
    xs = jnp.sum(xb * xb, axis=0)
    xsr = jnp.sum(xr * xr, axis=0)
    g = lax.dot_general(xr, xb, (((0,), (0,)), ((), ())),
                        preferred_element_type=jnp.float32)
    d = xsr[:, None] + xs[None, :] - 2.0 * g
    iota = lax.broadcasted_iota(jnp.int32, (RB, N), 1)
    off = b * N

    def body(t, carry):
        mn, acc = carry
        am = jnp.min(jnp.where(d == mn, iota, N), axis=1)
        mn_next = jnp.min(jnp.where(d > mn, d, jnp.inf), axis=1,
                          keepdims=True)
        acc = jnp.where(lane32 == t, (am + off)[:, None], acc)
        return mn_next, acc

    lane32 = lax.broadcasted_iota(jnp.int32, (RB, 32), 1)
    acc0 = jnp.zeros((RB, 32), jnp.int32)
    mn0 = jnp.min(d, axis=1, keepdims=True)
    _, acc = lax.fori_loop(0, K_NN + 1, body, (mn0, acc0))
    idx_ref[0] = acc.T[:K_NN + 1, :]

    u_ref[0] = lax.dot_general(xr, wd_ref[...], (((0,), (1,)), ((), ())),
                               preferred_element_type=jnp.float32) \
        + bias_ref[0][None, :]
    v_ref[0] = lax.dot_general(xr, w2_ref[...], (((0,), (1,)), ((), ())),
                               preferred_element_type=jnp.float32)


def _phase_a(x, wd, w2, bias, interpret=False):
    B, F, N = x.shape
    Fout = wd.shape[0]
    return pl.pallas_call(
        _phase_a_body,
        grid=(B, N // RB),
        in_specs=[
            pl.BlockSpec((1, F, N), lambda b, i: (b, 0, 0)),
            pl.BlockSpec((Fout, F), lambda b, i: (0, 0)),
            pl.BlockSpec((Fout, F), lambda b, i: (0, 0)),
            pl.BlockSpec((1, Fout), lambda b, i: (0, 0)),
        ],
        out_specs=[
            pl.BlockSpec((1, K_NN + 1, RB), lambda b, i: (b, 0, i)),
            pl.BlockSpec((1, RB, Fout), lambda b, i: (b, i, 0)),
            pl.BlockSpec((1, RB, Fout), lambda b, i: (b, i, 0)),
        ],
        out_shape=[
            jax.ShapeDtypeStruct((B, K_NN + 1, N), jnp.int32),
            jax.ShapeDtypeStruct((B, N, Fout), jnp.float32),
            jax.ShapeDtypeStruct((B, N, Fout), jnp.float32),
        ],
        interpret=interpret,
    )(x, wd, w2, bias)


_SC_CORES = 2
_SC_SUBCORES = 16
_NW = _SC_CORES * _SC_SUBCORES
_CH = 64


def _phase_b(vg, idxg):
    BN, F = vg.shape
    B, K1, N = idxg.shape
    npw = BN // _NW
    nchunk = npw // _CH
    mesh = plsc.VectorSubcoreMesh(core_axis_name="c", subcore_axis_name="s")

    @functools.partial(
        pl.kernel,
        out_type=[jax.ShapeDtypeStruct((BN, F), jnp.float32)] * 3,
        mesh=mesh,
        scratch_types=[
            pltpu.VMEM((K1, npw), jnp.int32),
            pltpu.VMEM((K_NN, _CH, F), jnp.float32),
            pltpu.VMEM((_CH, F), jnp.float32),
            pltpu.VMEM((_CH, F), jnp.float32),
            pltpu.VMEM((_CH, F), jnp.float32),
            pltpu.SemaphoreType.DMA,
        ],
        compiler_params=pltpu.CompilerParams(use_tc_tiling_on_sc=False),
    )
    def kern(vg_hbm, idx_hbm, mx_hbm, sm_hbm, sq_hbm,
             idxb, rows, om, osum, osq, sem):
        wid = lax.axis_index("s") * _SC_CORES + lax.axis_index("c")
        g0 = wid * npw
        bb = g0 // N
        n0 = g0 % N
        pltpu.sync_copy(idx_hbm.at[bb, :, pl.ds(n0, npw)], idxb)
        for chunk in range(nchunk):
            base = chunk * _CH
            descs = [
                pltpu.async_copy(
                    vg_hbm.at[idxb.at[t + 1, pl.ds(base, _CH)]],
                    rows.at[t], sem)
                for t in range(K_NN)
            ]
            for dsc in descs:
                dsc.wait()

            def nnbody(nn, carry):
                for c in range(F // 16):
                    sl = pl.ds(c * 16, 16)
                    r = rows[0, nn, sl]
                    m = r
                    s = r
                    q = r * r
                    for t in range(1, K_NN):
                        r = rows[t, nn, sl]
                        m = jnp.maximum(m, r)
                        s = s + r
                        q = q + r * r
                    om[nn, sl] = m
                    osum[nn, sl] = s
                    osq[nn, sl] = q
                return carry

            lax.fori_loop(0, _CH, nnbody, 0)
            pltpu.sync_copy(om, mx_hbm.at[pl.ds(g0 + base, _CH)])
            pltpu.sync_copy(osum, sm_hbm.at[pl.ds(g0 + base, _CH)])
            pltpu.sync_copy(osq, sq_hbm.at[pl.ds(g0 + base, _CH)])

    return kern(vg, idxg)


def _phase_c1_body(u_ref, s_ref, q_ref, s1_ref, s2_ref):
    i = pl.program_id(0)

    @pl.when(i == 0)
    def _():
        s1_ref[...] = jnp.zeros_like(s1_ref)
        s2_ref[...] = jnp.zeros_like(s2_ref)

    u = u_ref[...]
    s = s_ref[...]
    q = q_ref[...]
    k = jnp.float32(K_NN)
    y1 = jnp.sum(k * u + s, axis=0)
    y2 = jnp.sum(k * (u * u) + 2.0 * (u * s) + q, axis=0)
    s1_ref[...] = s1_ref[...] + y1[None, :]
    s2_ref[...] = s2_ref[...] + y2[None, :]


def _phase_c1(u2, sm, sq, interpret=False):
    BN, F = u2.shape
    return pl.pallas_call(
        _phase_c1_body,
        grid=(BN // RB,),
        in_specs=[pl.BlockSpec((RB, F), lambda i: (i, 0))] * 3,
        out_specs=[pl.BlockSpec((1, F), lambda i: (0, 0))] * 2,
        out_shape=[jax.ShapeDtypeStruct((1, F), jnp.float32)] * 2,
        interpret=interpret,
    )(u2, sm, sq)


def _phase_c2_body(n_total, u_ref, mx_ref, s1_ref, s2_ref, g_ref, be_ref,
                   out_ref):
    mean = s1_ref[0] / n_total
    var = s2_ref[0] / n_total - mean * mean
    inv = lax.rsqrt(var + 1e-5)
    scale = g_ref[0] * inv
    z = (u_ref[0] + mx_ref[...] - mean[None, :]) * scale[None, :] \
        + be_ref[0][None, :]
    z = jnp.maximum(z, 0.0)
    out_ref[0] = z.T


def _phase_c2(u, mx, s1, s2, gamma, beta, interpret=False):
    B, N, F = u.shape
    n_total = float(B * N * K_NN)
    return pl.pallas_call(
        functools.partial(_phase_c2_body, n_total),
        grid=(B, N // RB),
        in_specs=[
            pl.BlockSpec((1, RB, F), lambda b, i: (b, i, 0)),
            pl.BlockSpec((RB, F), lambda b, i: (b * (N // RB) + i, 0)),
            pl.BlockSpec((1, F), lambda b, i: (0, 0)),
            pl.BlockSpec((1, F), lambda b, i: (0, 0)),
            pl.BlockSpec((1, F), lambda b, i: (0, 0)),
            pl.BlockSpec((1, F), lambda b, i: (0, 0)),
        ],
        out_specs=pl.BlockSpec((1, F, RB), lambda b, i: (b, 0, i)),
        out_shape=jax.ShapeDtypeStruct((B, F, N), jnp.float32),
        interpret=interpret,
    )(u, mx, s1, s2, gamma, beta)


@jax.jit
def kernel(x, W, b, gamma, beta):
    B, F, N = x.shape
    Fout = W.shape[0]
    w1 = W[:, :F]
    w2 = W[:, F:]
    wd = w1 - w2
    idxg, u, v = _phase_a(x, wd, w2, b.reshape(1, Fout))
    vg = v.reshape(B * N, Fout)
    mx, sm, sq = _phase_b(vg, idxg)
    s1, s2 = _phase_c1(u.reshape(B * N, Fout), sm, sq)
    return _phase_c2(u, mx, s1, s2,
                     gamma.reshape(1, Fout), beta.reshape(1, Fout))
